# Initial kernel scaffold; baseline (speedup 1.0000x reference)
import jax
import jax.numpy as jnp
from jax.experimental import pallas as pl


def kernel(x, edge_index0, edge_index1, roots, type, W1a, b1a, W2a, b2a, W1b, b1b, W2b, b2b):
    raise NotImplementedError("write your pallas kernel here")


# speedup vs baseline: 3.3787x; 3.3787x over previous
import functools

import jax
import jax.numpy as jnp
from jax import lax
from jax.experimental import pallas as pl
from jax.experimental.pallas import tpu as pltpu
from jax.experimental.pallas import tpu_sc as plsc

N_NODES = 10000
N_EDGES = 160000
D_FEAT = 256
HIDDEN = 256
N_CLASSES = 40
N_ROOTS = 1024

NC = 2
NS = 16
HALF = 128

EPT = N_EDGES // NS
CH = 80
NCH = EPT // CH
NZC = N_NODES // CH
RPT = N_ROOTS // NS

_mesh = plsc.VectorSubcoreMesh(
    core_axis_name="c", subcore_axis_name="s", num_cores=NC, num_subcores=NS
)
_sc_params = pltpu.CompilerParams(use_tc_tiling_on_sc=False)


def _zero_vmem_2d(ref, rows, cols):
    zv = jnp.zeros((16,), jnp.float32)

    def body(r, _):
        for q in range(cols // 16):
            ref[r, pl.ds(q * 16, 16)] = zv
        return 0

    lax.fori_loop(0, rows, body, 0)


def _edge_phase(x2, src, dst, acc, cnta, srcv, gidxv, dstv, rowsv, onesv, sem,
                c, s, count):
    ebase = s * EPT

    def chunk(j, _):
        off = ebase + j * CH
        pltpu.sync_copy(src.at[pl.ds(off, CH)], srcv)
        pltpu.sync_copy(dst.at[pl.ds(off, CH)], dstv)
        for q in range(CH // 16):
            v = srcv[pl.ds(q * 16, 16)]
            gidxv[pl.ds(q * 16, 16)] = v * 2 + c
        pltpu.async_copy(x2.at[gidxv], rowsv, sem).wait()
        pltpu.sync_copy(rowsv, acc.at[dstv], add=True)
        if count:
            @pl.when(c == 0)
            def _():
                pltpu.sync_copy(onesv, cnta.at[dstv], add=True)
        return 0

    lax.fori_loop(0, NCH, chunk, 0)


def _foreach_row_chunk(s, fn):
    def body(i, _):
        j = s + i * NS

        @pl.when(j < NZC)
        def _():
            fn(j * CH)

        return 0

    lax.fori_loop(0, (NZC + NS - 1) // NS, body, 0)


def _init_phase(acc, cnta, rowsv, onesv, cstagev, c, s):
    _zero_vmem_2d(rowsv, CH, HALF)
    _zero_vmem_2d(cstagev, CH, 16)

    def zc(r0):
        pltpu.sync_copy(rowsv, acc.at[pl.ds(r0, CH)])

        @pl.when(c == 0)
        def _():
            pltpu.sync_copy(cstagev, cnta.at[pl.ds(r0, CH)])

    _foreach_row_chunk(s, zc)
    one0 = jnp.where(lax.iota(jnp.int32, 16) == 0, 1.0, 0.0)

    def ob(r, _):
        onesv[r, pl.ds(0, 16)] = one0
        return 0

    lax.fori_loop(0, CH, ob, 0)


@functools.partial(
    pl.kernel,
    out_type=(
        jax.ShapeDtypeStruct((NC, N_NODES, HALF), jnp.float32),
        jax.ShapeDtypeStruct((N_NODES, 16), jnp.float32),
    ),
    mesh=_mesh,
    scratch_types=[
        pltpu.VMEM_SHARED((N_NODES, HALF), jnp.float32),
        pltpu.VMEM_SHARED((N_NODES, 16), jnp.float32),
        pltpu.VMEM((CH,), jnp.int32),
        pltpu.VMEM((CH,), jnp.int32),
        pltpu.VMEM((CH,), jnp.int32),
        pltpu.VMEM((CH, HALF), jnp.float32),
        pltpu.VMEM((CH, 16), jnp.float32),
        pltpu.VMEM((CH, 16), jnp.float32),
        pltpu.SemaphoreType.DMA,
    ],
    compiler_params=_sc_params,
)
def _sc_segsum1(x2, src, dst, s1_out, cnt_out,
                acc, cnta, srcv, gidxv, dstv, rowsv, onesv, cstagev, sem):
    c = lax.axis_index("c")
    s = lax.axis_index("s")
    _init_phase(acc, cnta, rowsv, onesv, cstagev, c, s)
    plsc.subcore_barrier()
    _edge_phase(x2, src, dst, acc, cnta, srcv, gidxv, dstv, rowsv, onesv, sem,
                c, s, count=True)
    plsc.subcore_barrier()

    def wb(r0):
        pltpu.sync_copy(acc.at[pl.ds(r0, CH)], rowsv)
        pltpu.sync_copy(rowsv, s1_out.at[c, pl.ds(r0, CH)])

        @pl.when(c == 0)
        def _():
            pltpu.sync_copy(cnta.at[pl.ds(r0, CH)], cstagev)
            pltpu.sync_copy(cstagev, cnt_out.at[pl.ds(r0, CH)])

    _foreach_row_chunk(s, wb)


@functools.partial(
    pl.kernel,
    out_type=(
        jax.ShapeDtypeStruct((NC, N_ROOTS, HALF), jnp.float32),
        jax.ShapeDtypeStruct((N_ROOTS, 16), jnp.float32),
        jax.ShapeDtypeStruct((NC, N_ROOTS, HALF), jnp.float32),
    ),
    mesh=_mesh,
    scratch_types=[
        pltpu.VMEM_SHARED((N_NODES, HALF), jnp.float32),
        pltpu.VMEM_SHARED((N_NODES, 16), jnp.float32),
        pltpu.VMEM((CH,), jnp.int32),
        pltpu.VMEM((CH,), jnp.int32),
        pltpu.VMEM((CH,), jnp.int32),
        pltpu.VMEM((CH, HALF), jnp.float32),
        pltpu.VMEM((CH, 16), jnp.float32),
        pltpu.VMEM((CH, 16), jnp.float32),
        pltpu.VMEM((RPT,), jnp.int32),
        pltpu.VMEM((RPT,), jnp.int32),
        pltpu.VMEM((RPT, HALF), jnp.float32),
        pltpu.VMEM((RPT, 16), jnp.float32),
        pltpu.SemaphoreType.DMA,
    ],
    compiler_params=_sc_params,
)
def _sc_segsum2(h2, src, dst, roots, s0r_out, cntr_out, hr_out,
                acc, cnta, srcv, gidxv, dstv, rowsv, onesv, cstagev,
                rootv, rgidxv, rrowsv, rcntv, sem):
    c = lax.axis_index("c")
    s = lax.axis_index("s")
    _init_phase(acc, cnta, rowsv, onesv, cstagev, c, s)
    plsc.subcore_barrier()
    _edge_phase(h2, src, dst, acc, cnta, srcv, gidxv, dstv, rowsv, onesv, sem,
                c, s, count=True)
    plsc.subcore_barrier()
    rbase = s * RPT
    pltpu.sync_copy(roots.at[pl.ds(rbase, RPT)], rootv)
    pltpu.async_copy(acc.at[rootv], rrowsv, sem).wait()
    pltpu.sync_copy(rrowsv, s0r_out.at[c, pl.ds(rbase, RPT)])
    for q in range(RPT // 16):
        v = rootv[pl.ds(q * 16, 16)]
        rgidxv[pl.ds(q * 16, 16)] = v * 2 + c
    pltpu.async_copy(h2.at[rgidxv], rrowsv, sem).wait()
    pltpu.sync_copy(rrowsv, hr_out.at[c, pl.ds(rbase, RPT)])

    @pl.when(c == 0)
    def _():
        pltpu.async_copy(cnta.at[rootv], rcntv, sem).wait()
        pltpu.sync_copy(rcntv, cntr_out.at[pl.ds(rbase, RPT)])


def _tc1_body(x_ref, s1a_ref, s1b_ref, cnt_ref, w1_ref, w2a_ref, w2b_ref,
              b1_ref, b2_ref, o_ref):
    cnt = cnt_ref[:, 0:1]
    inv = 1.0 / jnp.maximum(cnt, 1.0)
    nz = jnp.where(cnt > 0.0, 1.0, 0.0)
    acc = jnp.dot(x_ref[...], w1_ref[...], preferred_element_type=jnp.float32)
    acc += jnp.dot(s1a_ref[...] * inv, w2a_ref[...],
                   preferred_element_type=jnp.float32)
    acc += jnp.dot(s1b_ref[...] * inv, w2b_ref[...],
                   preferred_element_type=jnp.float32)
    acc += b1_ref[...] + nz * b2_ref[...]
    o_ref[...] = jnp.maximum(acc, 0.0)


def _tc2_body(s0a_ref, s0b_ref, cnt_ref, ha_ref, hb_ref, w1a_ref, w1b_ref,
              w2a_ref, w2b_ref, b1_ref, b2_ref, o_ref):
    cnt = cnt_ref[:, 0:1]
    inv = 1.0 / jnp.maximum(cnt, 1.0)
    nz = jnp.where(cnt > 0.0, 1.0, 0.0)
    logits = jnp.dot(s0a_ref[...] * inv, w2a_ref[...],
                     preferred_element_type=jnp.float32)
    logits += jnp.dot(s0b_ref[...] * inv, w2b_ref[...],
                      preferred_element_type=jnp.float32)
    logits += jnp.dot(ha_ref[...], w1a_ref[...],
                      preferred_element_type=jnp.float32)
    logits += jnp.dot(hb_ref[...], w1b_ref[...],
                      preferred_element_type=jnp.float32)
    logits += b1_ref[...] + nz * b2_ref[...]
    m = jnp.max(logits, axis=1, keepdims=True)
    e = jnp.exp(logits - m)
    lse = jnp.log(jnp.sum(e, axis=1, keepdims=True))
    o_ref[...] = logits - m - lse


_BR1 = 128
_BR2 = 128


def _full(shape):
    return pl.BlockSpec(shape, lambda i: (0,) * len(shape))


def kernel(x, edge_index0, edge_index1, roots, type, W1a, b1a, W2a, b2a,
           W1b, b1b, W2b, b2b):
    del type
    src1 = edge_index1[0].astype(jnp.int32)
    dst1 = edge_index1[1].astype(jnp.int32)
    src0 = edge_index0[0].astype(jnp.int32)
    dst0 = edge_index0[1].astype(jnp.int32)
    roots_i = roots.astype(jnp.int32)
    x2 = x.reshape(2 * N_NODES, HALF)

    s1_halves, cnt1 = _sc_segsum1(x2, src1, dst1)

    grid1 = (N_NODES + _BR1 - 1) // _BR1
    h = pl.pallas_call(
        _tc1_body,
        grid=(grid1,),
        in_specs=[
            pl.BlockSpec((_BR1, D_FEAT), lambda i: (i, 0)),
            pl.BlockSpec((_BR1, HALF), lambda i: (i, 0)),
            pl.BlockSpec((_BR1, HALF), lambda i: (i, 0)),
            pl.BlockSpec((_BR1, 16), lambda i: (i, 0)),
            _full((D_FEAT, HIDDEN)),
            _full((HALF, HIDDEN)),
            _full((HALF, HIDDEN)),
            _full((1, HIDDEN)),
            _full((1, HIDDEN)),
        ],
        out_specs=pl.BlockSpec((_BR1, HIDDEN), lambda i: (i, 0)),
        out_shape=jax.ShapeDtypeStruct((N_NODES, HIDDEN), jnp.float32),
    )(x, s1_halves[0], s1_halves[1], cnt1, W1a, W2a[:HALF], W2a[HALF:],
      b1a.reshape(1, HIDDEN), b2a.reshape(1, HIDDEN))

    h2 = h.reshape(2 * N_NODES, HALF)
    s0r, cnt0r, hr = _sc_segsum2(h2, src0, dst0, roots_i)

    grid2 = N_ROOTS // _BR2
    out = pl.pallas_call(
        _tc2_body,
        grid=(grid2,),
        in_specs=[
            pl.BlockSpec((_BR2, HALF), lambda i: (i, 0)),
            pl.BlockSpec((_BR2, HALF), lambda i: (i, 0)),
            pl.BlockSpec((_BR2, 16), lambda i: (i, 0)),
            pl.BlockSpec((_BR2, HALF), lambda i: (i, 0)),
            pl.BlockSpec((_BR2, HALF), lambda i: (i, 0)),
            _full((HALF, N_CLASSES)),
            _full((HALF, N_CLASSES)),
            _full((HALF, N_CLASSES)),
            _full((HALF, N_CLASSES)),
            _full((1, N_CLASSES)),
            _full((1, N_CLASSES)),
        ],
        out_specs=pl.BlockSpec((_BR2, N_CLASSES), lambda i: (i, 0)),
        out_shape=jax.ShapeDtypeStruct((N_ROOTS, N_CLASSES), jnp.float32),
    )(s0r[0], s0r[1], cnt0r, hr[0], hr[1], W1b[:HALF], W1b[HALF:],
      W2b[:HALF], W2b[HALF:], b1b.reshape(1, N_CLASSES),
      b2b.reshape(1, N_CLASSES))
    return out


# speedup vs baseline: 7.6503x; 2.2643x over previous
import functools

import jax
import jax.numpy as jnp
from jax import lax
from jax.experimental import pallas as pl
from jax.experimental.pallas import tpu as pltpu
from jax.experimental.pallas import tpu_sc as plsc

N_NODES = 10000
N_EDGES = 160000
D_FEAT = 256
HIDDEN = 256
N_CLASSES = 40
N_ROOTS = 1024

NC = 2
NS = 16
HALF = 128

ECH1 = 80
EROWS1 = N_EDGES // ECH1
ERT1 = EROWS1 // NS
SUP1 = 25
NSUP1 = ERT1 // SUP1
ECH = 128
EROWS = N_EDGES // ECH
CW = 8
CH = 80
NZC = N_NODES // CH
RPT = N_ROOTS // NS

_mesh = plsc.VectorSubcoreMesh(
    core_axis_name="c", subcore_axis_name="s", num_cores=NC, num_subcores=NS
)
_sc_params = pltpu.CompilerParams(use_tc_tiling_on_sc=False)


def _zero_vmem_2d(ref, rows, cols):
    zv = jnp.zeros((16,), jnp.float32)

    def body(r, _):
        for q in range(cols // 16):
            ref[r, pl.ds(q * 16, 16)] = zv
        return 0

    lax.fori_loop(0, rows, body, 0)


def _edge_phase(x2, src2d, dst2d, acc, cnta, gidxv, dstv, rowsa, rowsb, onesv,
                sema, semb, c, s, count):
    def start(r, buf, sem):
        pltpu.async_copy(x2.at[gidxv.at[r]], buf, sem)

    def wait(buf, sem):
        pltpu.make_async_copy(x2.at[gidxv.at[0]], buf, sem).wait()

    def scat(buf, r):
        pltpu.sync_copy(buf, acc.at[dstv.at[r]], add=True)
        if count:
            @pl.when(c == 0)
            def _():
                pltpu.sync_copy(onesv, cnta.at[dstv.at[r]], add=True)

    def sup(k, _):
        base = ERT1 * s + k * SUP1
        pltpu.sync_copy(src2d.at[pl.ds(base, SUP1)], gidxv)
        pltpu.sync_copy(dst2d.at[pl.ds(base, SUP1)], dstv)

        def gb(r, _):
            for q in range(ECH1 // 16):
                v = gidxv[r, pl.ds(q * 16, 16)]
                gidxv[r, pl.ds(q * 16, 16)] = v * 2 + c
            return 0

        lax.fori_loop(0, SUP1, gb, 0)
        start(0, rowsa, sema)

        def body(r2, _):
            r = r2 * 2
            wait(rowsa, sema)
            start(r + 1, rowsb, semb)
            scat(rowsa, r)
            wait(rowsb, semb)

            @pl.when(r2 < SUP1 // 2 - 1)
            def _():
                start(r + 2, rowsa, sema)

            scat(rowsb, r + 1)
            return 0

        lax.fori_loop(0, SUP1 // 2, body, 0)
        start(SUP1 - 1, rowsa, sema)
        wait(rowsa, sema)
        scat(rowsa, SUP1 - 1)
        return 0

    lax.fori_loop(0, NSUP1, sup, 0)


def _foreach_row_chunk(s, fn):
    def body(i, _):
        j = s + i * NS

        @pl.when(j < NZC)
        def _():
            fn(j * CH)

        return 0

    lax.fori_loop(0, (NZC + NS - 1) // NS, body, 0)


def _init_phase(acc, cnta, rowsa, cstagev, onesv, c, s):
    _zero_vmem_2d(rowsa, CH, HALF)
    _zero_vmem_2d(cstagev, CH, 16)

    def zc(r0):
        pltpu.sync_copy(rowsa.at[pl.ds(0, CH)], acc.at[pl.ds(r0, CH)])

        @pl.when(c == 0)
        def _():
            pltpu.sync_copy(cstagev, cnta.at[pl.ds(r0, CH)])

    _foreach_row_chunk(s, zc)
    one0 = jnp.where(lax.iota(jnp.int32, 16) == 0, 1.0, 0.0)

    def ob(r, _):
        onesv[r, pl.ds(0, 16)] = one0
        return 0

    lax.fori_loop(0, ECH1, ob, 0)


@functools.partial(
    pl.kernel,
    out_type=(
        jax.ShapeDtypeStruct((NC, N_NODES, HALF), jnp.float32),
        jax.ShapeDtypeStruct((N_NODES, 16), jnp.float32),
    ),
    mesh=_mesh,
    scratch_types=[
        pltpu.VMEM_SHARED((N_NODES, HALF), jnp.float32),
        pltpu.VMEM_SHARED((N_NODES, 16), jnp.float32),
        pltpu.VMEM((SUP1, ECH1), jnp.int32),
        pltpu.VMEM((SUP1, ECH1), jnp.int32),
        pltpu.VMEM((ECH1, HALF), jnp.float32),
        pltpu.VMEM((ECH1, HALF), jnp.float32),
        pltpu.VMEM((ECH1, 16), jnp.float32),
        pltpu.VMEM((CH, 16), jnp.float32),
        pltpu.SemaphoreType.DMA,
        pltpu.SemaphoreType.DMA,
    ],
    compiler_params=_sc_params,
)
def _sc_segsum1(x2, src2d, dst2d, s1_out, cnt_out,
                acc, cnta, gidxv, dstv, rowsa, rowsb, onesv, cstagev,
                sema, semb):
    c = lax.axis_index("c")
    s = lax.axis_index("s")
    _init_phase(acc, cnta, rowsa, cstagev, onesv, c, s)
    plsc.subcore_barrier()
    _edge_phase(x2, src2d, dst2d, acc, cnta, gidxv, dstv, rowsa, rowsb, onesv,
                sema, semb, c, s, count=True)
    plsc.subcore_barrier()

    def wb(r0):
        pltpu.sync_copy(acc.at[pl.ds(r0, CH)], rowsa.at[pl.ds(0, CH)])
        pltpu.sync_copy(rowsa.at[pl.ds(0, CH)], s1_out.at[c, pl.ds(r0, CH)])

        @pl.when(c == 0)
        def _():
            pltpu.sync_copy(cnta.at[pl.ds(r0, CH)], cstagev)
            pltpu.sync_copy(cstagev, cnt_out.at[pl.ds(r0, CH)])

    _foreach_row_chunk(s, wb)


ZW = 64
ER2 = EROWS // NC
ERT2 = ER2 // NS
WPT = N_ROOTS // (NC * NS)


@functools.partial(
    pl.kernel,
    out_type=(
        jax.ShapeDtypeStruct((NC, N_ROOTS, ZW), jnp.float32),
        jax.ShapeDtypeStruct((NC, N_ROOTS, 16), jnp.float32),
        jax.ShapeDtypeStruct((N_ROOTS, ZW), jnp.float32),
    ),
    mesh=_mesh,
    scratch_types=[
        pltpu.VMEM_SHARED((N_NODES, ZW), jnp.float32),
        pltpu.VMEM_SHARED((N_NODES, 16), jnp.float32),
        pltpu.VMEM((ERT2 + 1, ECH), jnp.int32),
        pltpu.VMEM((ERT2 + 1, ECH), jnp.int32),
        pltpu.VMEM((ECH, ZW), jnp.float32),
        pltpu.VMEM((ECH, ZW), jnp.float32),
        pltpu.VMEM((ECH, 16), jnp.float32),
        pltpu.VMEM((CH, 16), jnp.float32),
        pltpu.VMEM((RPT,), jnp.int32),
        pltpu.VMEM((RPT, ZW), jnp.float32),
        pltpu.VMEM((RPT, 16), jnp.float32),
        pltpu.VMEM((WPT,), jnp.int32),
        pltpu.VMEM((WPT, ZW), jnp.float32),
        pltpu.SemaphoreType.DMA,
        pltpu.SemaphoreType.DMA,
    ],
    compiler_params=_sc_params,
)
def _sc_segsum2(zw2, src2d, dst2d, roots, s0r_out, cntr_out, wr_out,
                acc, cnta, gidxv, dstv, rowsa, rowsb, onesv, cstagev,
                rootv, rrowsv, rcntv, wrootv, wrowsv, sema, semb):
    c = lax.axis_index("c")
    s = lax.axis_index("s")

    _zero_vmem_2d(rowsa, CH, ZW)
    _zero_vmem_2d(cstagev, CH, 16)

    def zc(r0):
        pltpu.sync_copy(rowsa.at[pl.ds(0, CH)], acc.at[pl.ds(r0, CH)])
        pltpu.sync_copy(cstagev, cnta.at[pl.ds(r0, CH)])

    _foreach_row_chunk(s, zc)
    one0 = jnp.where(lax.iota(jnp.int32, 16) == 0, 1.0, 0.0)

    def ob(r, _):
        onesv[r, pl.ds(0, 16)] = one0
        return 0

    lax.fori_loop(0, ECH, ob, 0)
    plsc.subcore_barrier()

    base = ER2 * c + ERT2 * s + jnp.minimum(s, 1)
    extra = s < 1
    pltpu.sync_copy(src2d.at[pl.ds(base, ERT2)], gidxv.at[pl.ds(0, ERT2)])
    pltpu.sync_copy(dst2d.at[pl.ds(base, ERT2)], dstv.at[pl.ds(0, ERT2)])

    @pl.when(extra)
    def _():
        pltpu.sync_copy(src2d.at[pl.ds(base + ERT2, 1)],
                        gidxv.at[pl.ds(ERT2, 1)])
        pltpu.sync_copy(dst2d.at[pl.ds(base + ERT2, 1)],
                        dstv.at[pl.ds(ERT2, 1)])

    nrows = jnp.where(extra, ERT2 + 1, ERT2)

    def gb(r, _):
        for q in range(ECH // 16):
            v = gidxv[r, pl.ds(q * 16, 16)]
            gidxv[r, pl.ds(q * 16, 16)] = v * 2
        return 0

    lax.fori_loop(0, nrows, gb, 0)

    def start(r, buf, sem):
        pltpu.async_copy(zw2.at[gidxv.at[r]], buf, sem)

    def wait(buf, sem):
        pltpu.make_async_copy(zw2.at[gidxv.at[0]], buf, sem).wait()

    def scat(buf, r):
        pltpu.sync_copy(buf, acc.at[dstv.at[r]], add=True)
        pltpu.sync_copy(onesv, cnta.at[dstv.at[r]], add=True)

    start(0, rowsa, sema)

    def body(r2, _):
        r = r2 * 2
        wait(rowsa, sema)
        start(r + 1, rowsb, semb)
        scat(rowsa, r)
        wait(rowsb, semb)

        @pl.when(r2 < ERT2 // 2 - 1)
        def _():
            start(r + 2, rowsa, sema)

        scat(rowsb, r + 1)
        return 0

    lax.fori_loop(0, ERT2 // 2, body, 0)
    start(ERT2 - 1, rowsa, sema)
    wait(rowsa, sema)

    @pl.when(extra)
    def _():
        start(ERT2, rowsb, semb)

    scat(rowsa, ERT2 - 1)

    @pl.when(extra)
    def _():
        wait(rowsb, semb)
        scat(rowsb, ERT2)

    plsc.subcore_barrier()

    rbase = s * RPT
    pltpu.sync_copy(roots.at[pl.ds(rbase, RPT)], rootv)
    pltpu.async_copy(acc.at[rootv], rrowsv, sema).wait()
    pltpu.sync_copy(rrowsv, s0r_out.at[c, pl.ds(rbase, RPT)])
    pltpu.async_copy(cnta.at[rootv], rcntv, sema).wait()
    pltpu.sync_copy(rcntv, cntr_out.at[c, pl.ds(rbase, RPT)])
    wbase = c * (N_ROOTS // NC) + s * WPT
    pltpu.sync_copy(roots.at[pl.ds(wbase, WPT)], wrootv)
    for q in range(WPT // 16):
        v = wrootv[pl.ds(q * 16, 16)]
        wrootv[pl.ds(q * 16, 16)] = v * 2 + 1
    pltpu.async_copy(zw2.at[wrootv], wrowsv, sema).wait()
    pltpu.sync_copy(wrowsv, wr_out.at[pl.ds(wbase, WPT)])


def _tc1_body(x_ref, s1a_ref, s1b_ref, cnt_ref, w1_ref, w2a_ref, w2b_ref,
              b1_ref, b2_ref, w2z_ref, w1z_ref, zw_ref):
    cnt = cnt_ref[:, 0:1]
    inv = 1.0 / jnp.maximum(cnt, 1.0)
    nz = jnp.where(cnt > 0.0, 1.0, 0.0)
    acc = jnp.dot(x_ref[...], w1_ref[...], preferred_element_type=jnp.float32)
    acc += jnp.dot(s1a_ref[...] * inv, w2a_ref[...],
                   preferred_element_type=jnp.float32)
    acc += jnp.dot(s1b_ref[...] * inv, w2b_ref[...],
                   preferred_element_type=jnp.float32)
    acc += b1_ref[...] + nz * b2_ref[...]
    h = jnp.maximum(acc, 0.0)
    z = jnp.dot(h, w2z_ref[...], preferred_element_type=jnp.float32)
    w = jnp.dot(h, w1z_ref[...], preferred_element_type=jnp.float32)
    zw_ref[...] = jnp.concatenate([z, w], axis=1)


def _tc2_body(s0_ref, cnt_ref, wr_ref, b1_ref, b2_ref, o_ref):
    cnt2 = cnt_ref[...]
    cnt = cnt2[0, :, 0:1] + cnt2[1, :, 0:1]
    inv = 1.0 / jnp.maximum(cnt, 1.0)
    nz = jnp.where(cnt > 0.0, 1.0, 0.0)
    s0 = s0_ref[...]
    logits = (s0[0] + s0[1]) * inv + wr_ref[...] + b1_ref[...] + nz * b2_ref[...]
    col = lax.broadcasted_iota(jnp.int32, logits.shape, 1)
    valid = col < N_CLASSES
    ml = jnp.where(valid, logits, -1e30)
    m = jnp.max(ml, axis=1, keepdims=True)
    e = jnp.where(valid, jnp.exp(ml - m), 0.0)
    lse = jnp.log(jnp.sum(e, axis=1, keepdims=True))
    o_ref[...] = ml - m - lse


_BR1 = 128
_BR2 = 128


def _full(shape):
    return pl.BlockSpec(shape, lambda i: (0,) * len(shape))


def kernel(x, edge_index0, edge_index1, roots, type, W1a, b1a, W2a, b2a,
           W1b, b1b, W2b, b2b):
    del type
    src1 = edge_index1[0].astype(jnp.int32).reshape(EROWS1, ECH1)
    dst1 = edge_index1[1].astype(jnp.int32).reshape(EROWS1, ECH1)
    src0 = edge_index0[0].astype(jnp.int32).reshape(EROWS, ECH)
    dst0 = edge_index0[1].astype(jnp.int32).reshape(EROWS, ECH)
    roots_i = roots.astype(jnp.int32)
    x2 = x.reshape(2 * N_NODES, HALF)
    pad = ((0, 0), (0, ZW - N_CLASSES))
    W2b64 = jnp.pad(W2b, pad)
    W1b64 = jnp.pad(W1b, pad)
    b1b64 = jnp.pad(b1b, (0, ZW - N_CLASSES)).reshape(1, ZW)
    b2b64 = jnp.pad(b2b, (0, ZW - N_CLASSES)).reshape(1, ZW)

    s1_halves, cnt1 = _sc_segsum1(x2, src1, dst1)

    grid1 = (N_NODES + _BR1 - 1) // _BR1
    zw = pl.pallas_call(
        _tc1_body,
        grid=(grid1,),
        in_specs=[
            pl.BlockSpec((_BR1, D_FEAT), lambda i: (i, 0)),
            pl.BlockSpec((_BR1, HALF), lambda i: (i, 0)),
            pl.BlockSpec((_BR1, HALF), lambda i: (i, 0)),
            pl.BlockSpec((_BR1, 16), lambda i: (i, 0)),
            _full((D_FEAT, HIDDEN)),
            _full((HALF, HIDDEN)),
            _full((HALF, HIDDEN)),
            _full((1, HIDDEN)),
            _full((1, HIDDEN)),
            _full((HIDDEN, ZW)),
            _full((HIDDEN, ZW)),
        ],
        out_specs=pl.BlockSpec((_BR1, 2 * ZW), lambda i: (i, 0)),
        out_shape=jax.ShapeDtypeStruct((N_NODES, 2 * ZW), jnp.float32),
    )(x, s1_halves[0], s1_halves[1], cnt1, W1a, W2a[:HALF], W2a[HALF:],
      b1a.reshape(1, HIDDEN), b2a.reshape(1, HIDDEN), W2b64, W1b64)

    zw2 = zw.reshape(2 * N_NODES, ZW)
    s0r, cnt0r, wr = _sc_segsum2(zw2, src0, dst0, roots_i)

    grid2 = N_ROOTS // _BR2
    out = pl.pallas_call(
        _tc2_body,
        grid=(grid2,),
        in_specs=[
            pl.BlockSpec((NC, _BR2, ZW), lambda i: (0, i, 0)),
            pl.BlockSpec((NC, _BR2, 16), lambda i: (0, i, 0)),
            pl.BlockSpec((_BR2, ZW), lambda i: (i, 0)),
            _full((1, ZW)),
            _full((1, ZW)),
        ],
        out_specs=pl.BlockSpec((_BR2, ZW), lambda i: (i, 0)),
        out_shape=jax.ShapeDtypeStruct((N_ROOTS, ZW), jnp.float32),
    )(s0r, cnt0r, wr, b1b64, b2b64)
    return out[:, :N_CLASSES]
